# Initial kernel scaffold; baseline (speedup 1.0000x reference)
import jax
import jax.numpy as jnp
from jax.experimental import pallas as pl


def kernel(x, edge_index, W1, b1, W2, b2, W3, b3):
    raise NotImplementedError("write your pallas kernel here")


# speedup vs baseline: 17.4551x; 17.4551x over previous
import functools

import jax
import jax.numpy as jnp
from jax import lax
from jax.experimental import pallas as pl
from jax.experimental.pallas import tpu as pltpu
from jax.experimental.pallas import tpu_sc as plsc

NC = 2
NS = 16
NW = NC * NS
CHUNK = 128
F = 128

_mesh = plsc.VectorSubcoreMesh(
    core_axis_name="c", subcore_axis_name="s", num_cores=NC, num_subcores=NS)


def _make_count(npad, nch):
  rows = npad // NS
  nblk = rows // CHUNK

  @functools.partial(
      pl.kernel,
      out_type=jax.ShapeDtypeStruct((NC, npad, F), jnp.float32),
      mesh=_mesh,
      scratch_types=[
          pltpu.VMEM((nch, CHUNK), jnp.int32),
          pltpu.VMEM((CHUNK, F), jnp.float32),
          pltpu.VMEM((CHUNK, F), jnp.float32),
          pltpu.VMEM_SHARED((npad, F), jnp.float32),
      ],
  )
  def count_kernel(dst_hbm, z_hbm, ones_hbm, out_hbm, idx_d, ones_v, buf, acc):
    c = lax.axis_index("c")
    s = lax.axis_index("s")
    wid = s * NC + c
    pltpu.sync_copy(z_hbm, buf)
    pltpu.sync_copy(ones_hbm, ones_v)
    pltpu.sync_copy(dst_hbm.at[wid], idx_d)

    @pl.loop(0, nblk)
    def _(k):
      pltpu.sync_copy(buf, acc.at[pl.ds(s * rows + k * CHUNK, CHUNK)])

    plsc.subcore_barrier()

    @pl.loop(0, nch)
    def _(j):
      pltpu.sync_copy(ones_v, acc.at[idx_d.at[j]], add=True)

    plsc.subcore_barrier()

    @pl.loop(0, nblk)
    def _(k):
      pltpu.sync_copy(acc.at[pl.ds(s * rows + k * CHUNK, CHUNK)], buf)
      pltpu.sync_copy(buf, out_hbm.at[c, pl.ds(s * rows + k * CHUNK, CHUNK)])

  return count_kernel


def _make_agg(npad, nch):
  rows = npad // NS
  nblk = rows // CHUNK

  @functools.partial(
      pl.kernel,
      out_type=jax.ShapeDtypeStruct((NC, npad, F), jnp.float32),
      mesh=_mesh,
      scratch_types=[
          pltpu.VMEM((nch, CHUNK), jnp.int32),
          pltpu.VMEM((nch, CHUNK), jnp.int32),
          pltpu.VMEM((CHUNK, F), jnp.float32),
          pltpu.VMEM_SHARED((npad, F), jnp.float32),
          pltpu.SemaphoreType.DMA,
      ],
  )
  def agg_kernel(hs_hbm, src_hbm, dst_hbm, z_hbm, out_hbm,
                 idx_s, idx_d, gbuf, acc, sem):
    c = lax.axis_index("c")
    s = lax.axis_index("s")
    wid = s * NC + c
    pltpu.sync_copy(z_hbm, gbuf)
    pltpu.sync_copy(src_hbm.at[wid], idx_s)
    pltpu.sync_copy(dst_hbm.at[wid], idx_d)

    @pl.loop(0, nblk)
    def _(k):
      pltpu.sync_copy(gbuf, acc.at[pl.ds(s * rows + k * CHUNK, CHUNK)])

    plsc.subcore_barrier()

    @pl.loop(0, nch)
    def _(j):
      pltpu.async_copy(hs_hbm.at[idx_s.at[j]], gbuf, sem).wait()
      pltpu.sync_copy(gbuf, acc.at[idx_d.at[j]], add=True)

    plsc.subcore_barrier()

    @pl.loop(0, nblk)
    def _(k):
      pltpu.sync_copy(acc.at[pl.ds(s * rows + k * CHUNK, CHUNK)], gbuf)
      pltpu.sync_copy(gbuf, out_hbm.at[c, pl.ds(s * rows + k * CHUNK, CHUNK)])

  return agg_kernel


def _tc1_body(x_ref, w_ref, cnt_ref, hs_ref, dinv_ref):
  deg = cnt_ref[0, :, :16] + cnt_ref[1, :, :16] + 1.0
  dinv = lax.rsqrt(deg)
  dinv_ref[...] = dinv
  h = jnp.dot(x_ref[...], w_ref[...], preferred_element_type=jnp.float32)
  hs_ref[...] = h * dinv[:, :1]


def _tc_mid_body(p_ref, hs_ref, dinv_ref, b_ref, w_ref, out_ref):
  agg = p_ref[0] + p_ref[1] + hs_ref[...]
  dinv = dinv_ref[:, :1]
  z = jnp.maximum(agg * dinv + b_ref[...], 0.0)
  h = jnp.dot(z, w_ref[...], preferred_element_type=jnp.float32)
  out_ref[...] = h * dinv


def _tc_out_body(p_ref, g_ref, dinv_ref, b_ref, out_ref):
  agg = p_ref[0] + p_ref[1] + g_ref[...]
  out_ref[...] = agg * dinv_ref[:, :1] + b_ref[...]


def kernel(x, edge_index, W1, b1, W2, b2, W3, b3):
  n, d = x.shape
  h = W1.shape[1]
  c_out = W3.shape[1]
  e = edge_index.shape[1]

  npad = ((n + 127) // 128 + 1) * 128
  ew = e // NW
  nch = (ew + CHUNK - 1) // CHUNK
  padlen = nch * CHUNK - ew

  src = edge_index[0].reshape(NW, ew)
  dst = edge_index[1].reshape(NW, ew)
  pad_idx = (n + (jnp.arange(padlen, dtype=jnp.int32) % 128))
  padb = jnp.broadcast_to(pad_idx, (NW, padlen))
  srcp = jnp.concatenate([src, padb], axis=1).reshape(NW, nch, CHUNK)
  dstp = jnp.concatenate([dst, padb], axis=1).reshape(NW, nch, CHUNK)

  xp = jnp.pad(x, ((0, npad - n), (0, 0)))
  zf = jnp.zeros((CHUNK, F), jnp.float32)
  onf = jnp.ones((CHUNK, F), jnp.float32)
  w1p = jnp.pad(W1, ((0, 0), (0, F - h)))
  w2p = jnp.pad(W2, ((0, F - h), (0, F - h)))
  w3p = jnp.pad(W3, ((0, F - h), (0, F - c_out)))
  b1r = jnp.pad(b1, (0, F - h)).reshape(1, F)
  b2r = jnp.pad(b2, (0, F - h)).reshape(1, F)
  b3r = jnp.pad(b3, (0, F - c_out)).reshape(1, F)

  count_k = _make_count(npad, nch)
  agg_k = _make_agg(npad, nch)

  cnt = count_k(dstp, zf, onf)
  hs1, dinv16 = pl.pallas_call(
      _tc1_body,
      out_shape=(jax.ShapeDtypeStruct((npad, F), jnp.float32),
                 jax.ShapeDtypeStruct((npad, 16), jnp.float32)),
  )(xp, w1p, cnt)
  p1 = agg_k(hs1, srcp, dstp, zf)
  hs2 = pl.pallas_call(
      _tc_mid_body,
      out_shape=jax.ShapeDtypeStruct((npad, F), jnp.float32),
  )(p1, hs1, dinv16, b1r, w2p)
  p2 = agg_k(hs2, srcp, dstp, zf)
  g = pl.pallas_call(
      _tc_mid_body,
      out_shape=jax.ShapeDtypeStruct((npad, F), jnp.float32),
  )(p2, hs2, dinv16, b2r, w3p)
  p3 = agg_k(g, srcp, dstp, zf)
  o = pl.pallas_call(
      _tc_out_body,
      out_shape=jax.ShapeDtypeStruct((npad, F), jnp.float32),
  )(p3, g, dinv16, b3r)
  return o[:n, :c_out]


# speedup vs baseline: 29.6794x; 1.7003x over previous
import functools

import jax
import jax.numpy as jnp
from jax import lax
from jax.experimental import pallas as pl
from jax.experimental.pallas import tpu as pltpu
from jax.experimental.pallas import tpu_sc as plsc

NC = 2
NS = 16
NW = NC * NS
CHUNK = 128

_mesh = plsc.VectorSubcoreMesh(
    core_axis_name="c", subcore_axis_name="s", num_cores=NC, num_subcores=NS)
_sc_params = pltpu.CompilerParams(use_tc_tiling_on_sc=False)


def _make_count(npad, nch, f):
  rows = npad // NS
  nblk = rows // CHUNK

  @functools.partial(
      pl.kernel,
      out_type=jax.ShapeDtypeStruct((NC, npad, f), jnp.float32),
      mesh=_mesh,
      compiler_params=_sc_params,
      scratch_types=[
          pltpu.VMEM((nch, CHUNK), jnp.int32),
          pltpu.VMEM((CHUNK, f), jnp.float32),
          pltpu.VMEM((CHUNK, f), jnp.float32),
          pltpu.VMEM_SHARED((npad, f), jnp.float32),
      ],
  )
  def count_kernel(dst_hbm, z_hbm, ones_hbm, out_hbm, idx_d, ones_v, buf, acc):
    c = lax.axis_index("c")
    s = lax.axis_index("s")
    wid = s * NC + c
    pltpu.sync_copy(z_hbm, buf)
    pltpu.sync_copy(ones_hbm, ones_v)
    pltpu.sync_copy(dst_hbm.at[wid], idx_d)

    @pl.loop(0, nblk)
    def _(k):
      pltpu.sync_copy(buf, acc.at[pl.ds(s * rows + k * CHUNK, CHUNK)])

    plsc.subcore_barrier()

    @pl.loop(0, nch)
    def _(j):
      pltpu.sync_copy(ones_v, acc.at[idx_d.at[j]], add=True)

    plsc.subcore_barrier()

    @pl.loop(0, nblk)
    def _(k):
      pltpu.sync_copy(acc.at[pl.ds(s * rows + k * CHUNK, CHUNK)], buf)
      pltpu.sync_copy(buf, out_hbm.at[c, pl.ds(s * rows + k * CHUNK, CHUNK)])

  return count_kernel


def _make_agg(npad, nch, f, pipelined=True):
  rows = npad // NS
  nblk = rows // CHUNK

  @functools.partial(
      pl.kernel,
      out_type=jax.ShapeDtypeStruct((NC, npad, f), jnp.float32),
      mesh=_mesh,
      compiler_params=_sc_params,
      scratch_types=[
          pltpu.VMEM((nch, CHUNK), jnp.int32),
          pltpu.VMEM((nch, CHUNK), jnp.int32),
          pltpu.VMEM((CHUNK, f), jnp.float32),
          pltpu.VMEM((CHUNK, f), jnp.float32),
          pltpu.VMEM_SHARED((npad, f), jnp.float32),
          pltpu.SemaphoreType.DMA,
          pltpu.SemaphoreType.DMA,
      ],
  )
  def agg_kernel(hs_hbm, src_hbm, dst_hbm, z_hbm, out_hbm,
                 idx_s, idx_d, gbuf0, gbuf1, acc, sem0, sem1):
    c = lax.axis_index("c")
    s = lax.axis_index("s")
    wid = s * NC + c
    pltpu.sync_copy(z_hbm, gbuf0)
    pltpu.sync_copy(src_hbm.at[wid], idx_s)
    pltpu.sync_copy(dst_hbm.at[wid], idx_d)

    @pl.loop(0, nblk)
    def _(k):
      pltpu.sync_copy(gbuf0, acc.at[pl.ds(s * rows + k * CHUNK, CHUNK)])

    plsc.subcore_barrier()

    if pipelined:
      pltpu.async_copy(hs_hbm.at[idx_s.at[0]], gbuf0, sem0)

      @pl.loop(0, nch // 2)
      def _(jj):
        j0 = 2 * jj
        pltpu.make_async_copy(hs_hbm.at[idx_s.at[j0]], gbuf0, sem0).wait()
        pltpu.async_copy(hs_hbm.at[idx_s.at[j0 + 1]], gbuf1, sem1)
        pltpu.sync_copy(gbuf0, acc.at[idx_d.at[j0]], add=True)
        pltpu.make_async_copy(hs_hbm.at[idx_s.at[j0 + 1]], gbuf1, sem1).wait()

        @pl.when(j0 + 2 < nch)
        def _():
          pltpu.async_copy(hs_hbm.at[idx_s.at[j0 + 2]], gbuf0, sem0)

        pltpu.sync_copy(gbuf1, acc.at[idx_d.at[j0 + 1]], add=True)
    else:
      @pl.loop(0, nch)
      def _(j):
        pltpu.async_copy(hs_hbm.at[idx_s.at[j]], gbuf0, sem0).wait()
        pltpu.sync_copy(gbuf0, acc.at[idx_d.at[j]], add=True)

    plsc.subcore_barrier()

    @pl.loop(0, nblk)
    def _(k):
      pltpu.sync_copy(acc.at[pl.ds(s * rows + k * CHUNK, CHUNK)], gbuf0)
      pltpu.sync_copy(gbuf0, out_hbm.at[c, pl.ds(s * rows + k * CHUNK, CHUNK)])

  return agg_kernel


def _tc1_body(x_ref, w_ref, cnt_ref, hs_ref, dinv_ref):
  deg = cnt_ref[0] + cnt_ref[1] + 1.0
  dinv = lax.rsqrt(deg)
  dinv_ref[...] = dinv
  h = jnp.dot(x_ref[...], w_ref[...], preferred_element_type=jnp.float32)
  hs_ref[...] = h * dinv[:, :1]


def _tc_mid_body(p_ref, hs_ref, dinv_ref, b_ref, w_ref, out_ref):
  agg = p_ref[0] + p_ref[1] + hs_ref[...]
  dinv = dinv_ref[:, :1]
  z = jnp.maximum(agg * dinv + b_ref[...], 0.0)
  h = jnp.dot(z, w_ref[...], preferred_element_type=jnp.float32)
  out_ref[...] = h * dinv


def _tc_out_body(p_ref, g_ref, dinv_ref, b_ref, out_ref):
  agg = p_ref[0] + p_ref[1] + g_ref[...]
  out_ref[...] = agg * dinv_ref[:, :1] + b_ref[...]


def kernel(x, edge_index, W1, b1, W2, b2, W3, b3):
  n, d = x.shape
  h = W1.shape[1]
  c_out = W3.shape[1]
  e = edge_index.shape[1]

  npad = ((n + 127) // 128 + 1) * 128
  ew = e // NW
  nch = (ew + CHUNK - 1) // CHUNK
  nch += nch % 2
  padlen = nch * CHUNK - ew

  src = edge_index[0].reshape(NW, ew)
  dst = edge_index[1].reshape(NW, ew)
  pad_idx = (n + (jnp.arange(padlen, dtype=jnp.int32) % 128))
  padb = jnp.broadcast_to(pad_idx, (NW, padlen))
  srcp = jnp.concatenate([src, padb], axis=1).reshape(NW, nch, CHUNK)
  dstp = jnp.concatenate([dst, padb], axis=1).reshape(NW, nch, CHUNK)

  xp = jnp.pad(x, ((0, npad - n), (0, 0)))
  z16 = jnp.zeros((CHUNK, 16), jnp.float32)
  zh = jnp.zeros((CHUNK, h), jnp.float32)
  on16 = jnp.ones((CHUNK, 16), jnp.float32)
  w3p = jnp.pad(W3, ((0, 0), (0, 16 - c_out)))
  b1r = b1.reshape(1, h)
  b2r = b2.reshape(1, h)
  b3r = jnp.pad(b3, (0, 16 - c_out)).reshape(1, 16)

  count_k = _make_count(npad, nch, 16)
  agg_h = _make_agg(npad, nch, h)
  agg_o = _make_agg(npad, nch, 16)

  cnt = count_k(dstp, z16, on16)
  hs1, dinv16 = pl.pallas_call(
      _tc1_body,
      out_shape=(jax.ShapeDtypeStruct((npad, h), jnp.float32),
                 jax.ShapeDtypeStruct((npad, 16), jnp.float32)),
  )(xp, W1, cnt)
  p1 = agg_h(hs1, srcp, dstp, zh)
  hs2 = pl.pallas_call(
      _tc_mid_body,
      out_shape=jax.ShapeDtypeStruct((npad, h), jnp.float32),
  )(p1, hs1, dinv16, b1r, W2)
  p2 = agg_h(hs2, srcp, dstp, zh)
  g = pl.pallas_call(
      _tc_mid_body,
      out_shape=jax.ShapeDtypeStruct((npad, 16), jnp.float32),
  )(p2, hs2, dinv16, b2r, w3p)
  p3 = agg_o(g, srcp, dstp, z16)
  o = pl.pallas_call(
      _tc_out_body,
      out_shape=jax.ShapeDtypeStruct((npad, 16), jnp.float32),
  )(p3, g, dinv16, b3r)
  return o[:n, :c_out]
